# baseline (speedup 1.0000x reference)
import functools

import jax
import jax.numpy as jnp
from jax import lax
from jax.experimental import pallas as pl
from jax.experimental.pallas import tpu as pltpu
from jax.experimental.pallas import tpu_sc as plsc


def _gather_fn(B, D, b_per_w, NC):
    mesh = plsc.VectorSubcoreMesh(core_axis_name="c", subcore_axis_name="s")

    @functools.partial(
        pl.kernel,
        mesh=mesh,
        out_type=jax.ShapeDtypeStruct((B, D), jnp.float32),
        scratch_types=[
            pltpu.VMEM((b_per_w,), jnp.int32),
            pltpu.VMEM((b_per_w, D), jnp.float32),
            pltpu.SemaphoreType.DMA,
        ],
        compiler_params=pltpu.CompilerParams(use_tc_tiling_on_sc=False),
    )
    def gather_kernel(idx_hbm, table_hbm, out_hbm, idx_v, rows_v, sem):
        wid = lax.axis_index("s") * NC + lax.axis_index("c")
        base = wid * b_per_w
        pltpu.sync_copy(idx_hbm.at[pl.ds(base, b_per_w)], idx_v)
        pltpu.async_copy(table_hbm.at[idx_v], rows_v, sem).wait()
        pltpu.sync_copy(rows_v, out_hbm.at[pl.ds(base, b_per_w)])

    return gather_kernel


def kernel(inputs, table):
    B = inputs.shape[0]
    D = table.shape[1]
    info = plsc.get_sparse_core_info()
    NC, NS = info.num_cores, info.num_subcores
    NW = NC * NS
    assert B % NW == 0
    b_per_w = B // NW
    fn = _gather_fn(B, D, b_per_w, NC)
    return fn(inputs.astype(jnp.int32), table)


# speedup vs baseline: 1.4422x; 1.4422x over previous
import functools

import jax
import jax.numpy as jnp
from jax import lax
from jax.experimental import pallas as pl
from jax.experimental.pallas import tpu as pltpu
from jax.experimental.pallas import tpu_sc as plsc

_L = 16


def _gather_fn(B, D, b_per_w, NC):
    mesh = plsc.VectorSubcoreMesh(core_axis_name="c", subcore_axis_name="s")

    @functools.partial(
        pl.kernel,
        mesh=mesh,
        out_type=jax.ShapeDtypeStruct((B, D), jnp.float32),
        scratch_types=[
            pltpu.VMEM((b_per_w,), jnp.int32),
            pltpu.VMEM((b_per_w, D), jnp.float32),
            pltpu.SemaphoreType.DMA,
        ],
    )
    def gather_kernel(idx_hbm, table_hbm, out_hbm, idx_v, rows_v, sem):
        wid = lax.axis_index("s") * NC + lax.axis_index("c")
        base = wid * b_per_w
        pltpu.sync_copy(idx_hbm.at[pl.ds(base, b_per_w)], idx_v)
        for g in range(b_per_w // _L):
            iv = idx_v[pl.ds(g * _L, _L)]
            for l in range(_L):
                i = iv[l]
                pltpu.async_copy(
                    table_hbm.at[i], rows_v.at[g * _L + l], sem
                )
        pltpu.make_async_copy(
            table_hbm.at[pl.ds(0, b_per_w)], rows_v, sem
        ).wait()
        pltpu.sync_copy(rows_v, out_hbm.at[pl.ds(base, b_per_w)])

    return gather_kernel


def kernel(inputs, table):
    B = inputs.shape[0]
    D = table.shape[1]
    info = plsc.get_sparse_core_info()
    NC, NS = info.num_cores, info.num_subcores
    NW = NC * NS
    assert B % NW == 0
    b_per_w = B // NW
    fn = _gather_fn(B, D, b_per_w, NC)
    return fn(inputs.astype(jnp.int32), table)


# speedup vs baseline: 1.8392x; 1.2753x over previous
import functools

import jax
import jax.numpy as jnp
from jax import lax
from jax.experimental import pallas as pl
from jax.experimental.pallas import tpu as pltpu
from jax.experimental.pallas import tpu_sc as plsc

_L = 16
_PIECE = 512
_NPIECE = 7


def _splat(v, n=_L):
    return jnp.full((n,), v, jnp.int32)


def _gather_fn(B, V, D, NC, NW):
    mesh = plsc.VectorSubcoreMesh(core_axis_name="c", subcore_axis_name="s")
    TCOLS = V // 128
    TAIL = TCOLS * 128
    TAILW = V - TAIL
    CAP = B + _L
    MAXLO = TAIL - _PIECE

    @functools.partial(
        pl.kernel,
        mesh=mesh,
        out_type=jax.ShapeDtypeStruct((B, D), jnp.float32),
        scratch_types=[
            pltpu.VMEM((B,), jnp.int32),
            pltpu.VMEM((CAP,), jnp.int32),
            pltpu.VMEM((_NPIECE, CAP), jnp.int32),
            pltpu.VMEM((CAP,), jnp.int32),
            pltpu.VMEM((2, D, _PIECE), jnp.float32),
            pltpu.VMEM((D, TAILW), jnp.float32),
            pltpu.VMEM((4, _L, D), jnp.float32),
            pltpu.SemaphoreType.DMA,
            pltpu.SemaphoreType.DMA,
            pltpu.SemaphoreType.DMA,
            pltpu.SemaphoreType.DMA,
        ],
        compiler_params=pltpu.CompilerParams(needs_layout_passes=False),
    )
    def gather_kernel(
        idx_hbm, tabT_hbm, out_hbm,
        idx_all, inlist, plists, tlist, bufs, tailbuf, rowbufs,
        semA, semB, semT, semR,
    ):
        wid = lax.axis_index("s") * NC + lax.axis_index("c")
        lo_t = (wid * (TCOLS + 1)) // NW
        hi_t = ((wid + 1) * (TCOLS + 1)) // NW
        range_lo = lo_t * 128
        range_hi = hi_t * 128
        lanes = lax.iota(jnp.int32, _L)

        sems = [semA, semB]
        def piece_start(p):
            lo = pl.multiple_of(
                jnp.minimum(range_lo + p * _PIECE, MAXLO), 128
            )
            return (
                pltpu.async_copy(
                    tabT_hbm.at[:, pl.ds(lo, _PIECE)],
                    bufs.at[p % 2],
                    sems[p % 2],
                ),
                lo,
            )

        handles = {}
        handles[0] = piece_start(0)
        handles[1] = piece_start(1)
        hT = pltpu.async_copy(
            tabT_hbm.at[:, pl.ds(TAIL, TAILW)], tailbuf, semT
        )

        pltpu.sync_copy(idx_hbm, idx_all)

        def p1_body(g, c_in):
            iv = idx_all[pl.ds(g * _L, _L)]
            jv = lanes + g * _L
            pk = jnp.bitwise_or(lax.shift_left(jv, 17), iv)
            m = (iv >= range_lo) & (iv < range_hi)
            cs = plsc.cumsum(m.astype(jnp.int32))
            pos = c_in + cs - 1
            plsc.store_scatter(inlist, [pos], pk, mask=m)
            return c_in + cs[_L - 1]

        M_in = lax.fori_loop(0, B // _L, p1_body, jnp.int32(0), unroll=4)

        v0 = inlist[pl.ds(0, _L)]
        plsc.store_scatter(inlist, [M_in + lanes], _splat(0) + v0[0])

        ngrp_in = lax.shift_right_logical(M_in + (_L - 1), 4)

        def p2_body(m, carry):
            cs0, cs1, cs2, cs3, cs4, cs5, cs6, ct = carry
            pk = inlist[pl.ds(m * _L, _L)]
            iv = jnp.bitwise_and(pk, (1 << 17) - 1)
            q = lax.shift_right_logical(iv - range_lo, 9)
            m_tail = iv >= TAIL
            cursors = [cs0, cs1, cs2, cs3, cs4, cs5, cs6]
            new = []
            for p in range(_NPIECE):
                mp = (q == p) & jnp.logical_not(m_tail)
                cs = plsc.cumsum(mp.astype(jnp.int32))
                plsc.store_scatter(
                    plists, [_splat(p), cursors[p] + cs - 1], pk, mask=mp
                )
                new.append(cursors[p] + cs[_L - 1])
            cst = plsc.cumsum(m_tail.astype(jnp.int32))
            plsc.store_scatter(tlist, [ct + cst - 1], pk, mask=m_tail)
            return (*new, ct + cst[_L - 1])

        zero = jnp.int32(0)
        counts = lax.fori_loop(
            0, ngrp_in, p2_body, (zero,) * (_NPIECE + 1)
        )
        tail_cnt = counts[_NPIECE]

        def extract(load_pk, M, base, gather_col, G):
            ngrp = lax.shift_right_logical(M + (_L - 1), 4)

            def body(m, G):
                r = jnp.bitwise_and(G, 3)

                @pl.when(G >= 4)
                def _():
                    pltpu.make_async_copy(
                        out_hbm.at[pl.ds(0, _L)], rowbufs.at[r], semR
                    ).wait()

                pk = load_pk(m)
                jv = lax.shift_right_logical(pk, 17)
                iv = jnp.bitwise_and(pk, (1 << 17) - 1)
                il = iv - base
                rv = _splat(0) + r
                for c in range(D):
                    vals = gather_col(c, il)
                    plsc.store_scatter(
                        rowbufs, [rv, lanes, _splat(c)], vals
                    )
                for l in range(_L):
                    j = jv[l]
                    pltpu.async_copy(
                        rowbufs.at[r, l], out_hbm.at[j], semR
                    )
                return G + 1

            return lax.fori_loop(0, ngrp, body, G)

        G = jnp.int32(0)
        for p in range(_NPIECE):
            h, lo_p = handles[p]
            h.wait()
            cnt = counts[p]
            vp = plists[p, pl.ds(0, _L)]
            plsc.store_scatter(
                plists, [_splat(p), cnt + lanes], _splat(0) + vp[0]
            )
            k = p % 2

            def load_pk(m, _p=p):
                return plists[_p, pl.ds(m * _L, _L)]

            def gather_col(c, il, _k=k):
                return plsc.load_gather(
                    bufs, [_splat(_k), _splat(c), il]
                )

            G = extract(load_pk, cnt, lo_p, gather_col, G)
            if p + 2 < _NPIECE:
                handles[p + 2] = piece_start(p + 2)

        hT.wait()
        vp = tlist[pl.ds(0, _L)]
        plsc.store_scatter(tlist, [tail_cnt + lanes], _splat(0) + vp[0])

        def load_pk_t(m):
            return tlist[pl.ds(m * _L, _L)]

        def gather_col_t(c, il):
            return plsc.load_gather(tailbuf, [_splat(c), il])

        G = extract(load_pk_t, tail_cnt, jnp.int32(TAIL), gather_col_t, G)

        for kk in range(4):
            @pl.when(G > kk)
            def _(kk=kk):
                pltpu.make_async_copy(
                    out_hbm.at[pl.ds(0, _L)], rowbufs.at[kk], semR
                ).wait()

    return gather_kernel


def kernel(inputs, table):
    B = inputs.shape[0]
    V, D = table.shape
    info = plsc.get_sparse_core_info()
    NC, NS = info.num_cores, info.num_subcores
    NW = NC * NS
    assert B % _L == 0
    fn = _gather_fn(B, V, D, NC, NW)
    return fn(inputs.astype(jnp.int32), table.T)


# speedup vs baseline: 1.9596x; 1.0655x over previous
import functools

import jax
import jax.numpy as jnp
from jax import lax
from jax.experimental import pallas as pl
from jax.experimental.pallas import tpu as pltpu
from jax.experimental.pallas import tpu_sc as plsc

_L = 16
_PIECE = 512
_NPIECE = 7


def _splat(v, n=_L):
    return jnp.full((n,), v, jnp.int32)


def _gather_fn(B, V, D, NC, NW):
    mesh = plsc.VectorSubcoreMesh(core_axis_name="c", subcore_axis_name="s")
    TCOLS = V // 128
    TAIL = TCOLS * 128
    TAILW = V - TAIL
    CAP = B + _L
    MAXLO = TAIL - _PIECE

    @functools.partial(
        pl.kernel,
        mesh=mesh,
        out_type=jax.ShapeDtypeStruct((B, D), jnp.float32),
        scratch_types=[
            pltpu.VMEM((B,), jnp.int32),
            pltpu.VMEM((CAP,), jnp.int32),
            pltpu.VMEM((_NPIECE, CAP), jnp.int32),
            pltpu.VMEM((CAP,), jnp.int32),
            pltpu.VMEM((2, D, _PIECE), jnp.float32),
            pltpu.VMEM((D, TAILW), jnp.float32),
            pltpu.VMEM((4, _L, D), jnp.float32),
            pltpu.SemaphoreType.DMA,
            pltpu.SemaphoreType.DMA,
            pltpu.SemaphoreType.DMA,
            pltpu.SemaphoreType.DMA,
        ],
        compiler_params=pltpu.CompilerParams(needs_layout_passes=False),
    )
    def gather_kernel(
        idx_hbm, tabT_hbm, out_hbm,
        idx_all, inlist, plists, tlist, bufs, tailbuf, rowbufs,
        semA, semB, semT, semR,
    ):
        wid = lax.axis_index("s") * NC + lax.axis_index("c")
        lo_t = (wid * (TCOLS + 1)) // NW
        hi_t = ((wid + 1) * (TCOLS + 1)) // NW
        range_lo = lo_t * 128
        range_hi = hi_t * 128
        lanes = lax.iota(jnp.int32, _L)

        sems = [semA, semB]
        def piece_start(p):
            lo = pl.multiple_of(
                jnp.minimum(range_lo + p * _PIECE, MAXLO), 128
            )
            pltpu.async_copy(
                tabT_hbm.at[:, pl.ds(lo, _PIECE)],
                bufs.at[p % 2],
                sems[p % 2],
            )

        piece_start(0)
        piece_start(1)
        hT = pltpu.async_copy(
            tabT_hbm.at[:, pl.ds(TAIL, TAILW)], tailbuf, semT
        )

        pltpu.sync_copy(idx_hbm, idx_all)

        def p1_body(g, c_in):
            iv = idx_all[pl.ds(g * _L, _L)]
            jv = lanes + g * _L
            pk = jnp.bitwise_or(lax.shift_left(jv, 17), iv)
            m = (iv >= range_lo) & (iv < range_hi)
            cs = plsc.cumsum(m.astype(jnp.int32))
            pos = c_in + cs - 1
            plsc.store_scatter(inlist, [pos], pk, mask=m)
            return c_in + cs[_L - 1]

        M_in = lax.fori_loop(0, B // _L, p1_body, jnp.int32(0), unroll=4)

        v0 = inlist[pl.ds(0, _L)]
        plsc.store_scatter(inlist, [M_in + lanes], _splat(0) + v0[0])

        ngrp_in = lax.shift_right_logical(M_in + (_L - 1), 4)

        def p2_body(m, carry):
            cs0, cs1, cs2, cs3, cs4, cs5, cs6, ct = carry
            pk = inlist[pl.ds(m * _L, _L)]
            iv = jnp.bitwise_and(pk, (1 << 17) - 1)
            q = lax.shift_right_logical(iv - range_lo, 9)
            m_tail = iv >= TAIL
            cursors = [cs0, cs1, cs2, cs3, cs4, cs5, cs6]
            new = []
            for p in range(_NPIECE):
                mp = (q == p) & jnp.logical_not(m_tail)
                cs = plsc.cumsum(mp.astype(jnp.int32))
                plsc.store_scatter(
                    plists, [_splat(p), cursors[p] + cs - 1], pk, mask=mp
                )
                new.append(cursors[p] + cs[_L - 1])
            cst = plsc.cumsum(m_tail.astype(jnp.int32))
            plsc.store_scatter(tlist, [ct + cst - 1], pk, mask=m_tail)
            return (*new, ct + cst[_L - 1])

        zero = jnp.int32(0)
        counts = lax.fori_loop(
            0, ngrp_in, p2_body, (zero,) * (_NPIECE + 1)
        )
        tail_cnt = counts[_NPIECE]

        def extract(load_pk, M, base, gather_col, G):
            ngrp = lax.shift_right_logical(M + (_L - 1), 4)

            def body(m, G):
                r = jnp.bitwise_and(G, 3)

                @pl.when(G >= 4)
                def _():
                    pltpu.make_async_copy(
                        out_hbm.at[pl.ds(0, _L)], rowbufs.at[r], semR
                    ).wait()

                pk = load_pk(m)
                jv = lax.shift_right_logical(pk, 17)
                iv = jnp.bitwise_and(pk, (1 << 17) - 1)
                il = iv - base
                rv = _splat(0) + r
                for c in range(D):
                    vals = gather_col(c, il)
                    plsc.store_scatter(
                        rowbufs, [rv, lanes, _splat(c)], vals
                    )
                for l in range(_L):
                    j = jv[l]
                    pltpu.async_copy(
                        rowbufs.at[r, l], out_hbm.at[j], semR
                    )
                return G + 1

            return lax.fori_loop(0, ngrp, body, G)

        cntv = _splat(0)
        for p in range(_NPIECE):
            cntv = cntv + jnp.where(lanes == p, counts[p], 0)

        def piece_body(p, G):
            pv = _splat(0) + p
            M = jax.numpy.sum(jnp.where(lanes == p, cntv, 0))
            k = jnp.bitwise_and(p, 1)

            @pl.when(k == 0)
            def _():
                pltpu.make_async_copy(
                    tabT_hbm.at[:, pl.ds(0, _PIECE)], bufs.at[0], semA
                ).wait()

            @pl.when(k == 1)
            def _():
                pltpu.make_async_copy(
                    tabT_hbm.at[:, pl.ds(0, _PIECE)], bufs.at[1], semB
                ).wait()

            base = jnp.minimum(range_lo + p * _PIECE, MAXLO)
            vp = plsc.load_gather(plists, [pv, lanes])
            plsc.store_scatter(plists, [pv, M + lanes], _splat(0) + vp[0])
            kv = _splat(0) + k

            def load_pk(m):
                return plsc.load_gather(plists, [pv, m * _L + lanes])

            def gather_col(c, il):
                return plsc.load_gather(bufs, [kv, _splat(c), il])

            G = extract(load_pk, M, base, gather_col, G)

            @pl.when((p + 2 < _NPIECE) & (k == 0))
            def _():
                lo2 = pl.multiple_of(
                    jnp.minimum(range_lo + (p + 2) * _PIECE, MAXLO), 128
                )
                pltpu.async_copy(
                    tabT_hbm.at[:, pl.ds(lo2, _PIECE)], bufs.at[0], semA
                )

            @pl.when((p + 2 < _NPIECE) & (k == 1))
            def _():
                lo2 = pl.multiple_of(
                    jnp.minimum(range_lo + (p + 2) * _PIECE, MAXLO), 128
                )
                pltpu.async_copy(
                    tabT_hbm.at[:, pl.ds(lo2, _PIECE)], bufs.at[1], semB
                )

            return G

        G = lax.fori_loop(0, _NPIECE, piece_body, jnp.int32(0))

        hT.wait()
        vp = tlist[pl.ds(0, _L)]
        plsc.store_scatter(tlist, [tail_cnt + lanes], _splat(0) + vp[0])

        def load_pk_t(m):
            return tlist[pl.ds(m * _L, _L)]

        def gather_col_t(c, il):
            return plsc.load_gather(tailbuf, [_splat(c), il])

        G = extract(load_pk_t, tail_cnt, jnp.int32(TAIL), gather_col_t, G)

        for kk in range(4):
            @pl.when(G > kk)
            def _(kk=kk):
                pltpu.make_async_copy(
                    out_hbm.at[pl.ds(0, _L)], rowbufs.at[kk], semR
                ).wait()

    return gather_kernel


def kernel(inputs, table):
    B = inputs.shape[0]
    V, D = table.shape
    info = plsc.get_sparse_core_info()
    NC, NS = info.num_cores, info.num_subcores
    NW = NC * NS
    assert B % _L == 0
    fn = _gather_fn(B, V, D, NC, NW)
    return fn(inputs.astype(jnp.int32), table.T)
